# Initial kernel scaffold; baseline (speedup 1.0000x reference)
import jax
import jax.numpy as jnp
from jax.experimental import pallas as pl


def kernel(x, edge_index, batch, W1, b1, W2, b2, W3, b3, Wf1, bf1, Wf2, bf2):
    raise NotImplementedError("write your pallas kernel here")


# speedup vs baseline: 6.8668x; 6.8668x over previous
import functools

import jax
import jax.numpy as jnp
from jax import lax
from jax.experimental import pallas as pl
from jax.experimental.pallas import tpu as pltpu
from jax.experimental.pallas import tpu_sc as plsc

N = 10000
E = 320000
F_IN = 128
H = 256
HH = 128
G = 64
NC = 2
NS = 16
CH = 128
K = 160
GRP = 32
EPAD = NS * K * CH
NPAD = 10240
TRASH = N
RPT = NPAD // NS
BM = 400
NB = N // BM

_MESH = plsc.VectorSubcoreMesh(core_axis_name="c", subcore_axis_name="s",
                               num_cores=NC, num_subcores=NS)


@functools.partial(
    pl.kernel,
    out_type=(jax.ShapeDtypeStruct((NPAD,), jnp.float32),
              jax.ShapeDtypeStruct((NPAD,), jnp.float32)),
    mesh=_MESH,
    scratch_types=[
        pltpu.VMEM((K // NC, CH), jnp.int32),
        pltpu.VMEM((RPT,), jnp.float32),
        pltpu.VMEM((CH,), jnp.float32),
        pltpu.VMEM_SHARED((NPAD,), jnp.float32),
        pltpu.SemaphoreType.DMA,
    ],
)
def _deg_call(dst_hbm, deg_a, deg_b, idxv, zbuf, onev, dacc, sem):
    c = lax.axis_index("c")
    s = lax.axis_index("s")
    for i in range(RPT // 16):
        zbuf[pl.ds(i * 16, 16)] = jnp.zeros((16,), jnp.float32)
    for i in range(CH // 16):
        onev[pl.ds(i * 16, 16)] = jnp.ones((16,), jnp.float32)
    pltpu.sync_copy(zbuf, dacc.at[pl.ds(s * RPT, RPT)])
    pltpu.sync_copy(dst_hbm.at[s, pl.ds(c * (K // NC), K // NC)], idxv)
    plsc.subcore_barrier()

    def body(jj, carry):
        descs = []
        for b in range(8):
            k = jj * 8 + b
            descs.append(pltpu.async_copy(onev, dacc.at[idxv.at[k]], sem,
                                          add=True))
        for d in descs:
            d.wait()
        return carry

    lax.fori_loop(0, (K // NC) // 8, body, 0)
    plsc.subcore_barrier()

    @pl.when(c == 0)
    def _():
        pltpu.sync_copy(dacc.at[pl.ds(s * RPT, RPT)],
                        deg_a.at[pl.ds(s * RPT, RPT)])

    @pl.when(c == 1)
    def _():
        pltpu.sync_copy(dacc.at[pl.ds(s * RPT, RPT)],
                        deg_b.at[pl.ds(s * RPT, RPT)])


@functools.partial(
    pl.kernel,
    out_type=(jax.ShapeDtypeStruct((N, HH), jnp.float32),
              jax.ShapeDtypeStruct((N, HH), jnp.float32)),
    mesh=_MESH,
    scratch_types=[
        pltpu.VMEM((GRP, CH), jnp.int32),
        pltpu.VMEM((GRP, CH), jnp.int32),
        pltpu.VMEM((2, CH, HH), jnp.float32),
        pltpu.VMEM_SHARED((NPAD, HH), jnp.float32),
        pltpu.SemaphoreType.DMA,
        pltpu.SemaphoreType.DMA,
    ],
)
def _agg_call(hs_l, hs_r, src_hbm, dst_hbm, out_l, out_r,
              isrc, idst, rows, acc, sem_g, sem_s):
    c = lax.axis_index("c")
    s = lax.axis_index("s")

    def run(hs, out):
        @pl.when(s < NS - 1)
        def _():
            pltpu.sync_copy(hs.at[pl.ds(s * RPT, RPT)],
                            acc.at[pl.ds(s * RPT, RPT)])

        @pl.when(s == NS - 1)
        def _():
            pltpu.sync_copy(hs.at[pl.ds((NS - 1) * RPT, N - (NS - 1) * RPT)],
                            acc.at[pl.ds((NS - 1) * RPT, N - (NS - 1) * RPT)])

        plsc.subcore_barrier()

        def group(gi, gcarry):
            pltpu.sync_copy(src_hbm.at[s, pl.ds(gi * GRP, GRP)], isrc)
            pltpu.sync_copy(dst_hbm.at[s, pl.ds(gi * GRP, GRP)], idst)

            def body(jj, carry):
                k0 = 2 * jj
                g0 = pltpu.async_copy(hs.at[isrc.at[k0]], rows.at[0], sem_g)
                g1 = pltpu.async_copy(hs.at[isrc.at[k0 + 1]], rows.at[1],
                                      sem_g)
                g0.wait()
                s0 = pltpu.async_copy(rows.at[0], acc.at[idst.at[k0]], sem_s,
                                      add=True)
                g1.wait()
                s1 = pltpu.async_copy(rows.at[1], acc.at[idst.at[k0 + 1]],
                                      sem_s, add=True)
                s0.wait()
                s1.wait()
                return carry

            lax.fori_loop(0, GRP // 2, body, 0)
            return gcarry

        lax.fori_loop(0, K // GRP, group, 0)
        plsc.subcore_barrier()

        @pl.when(s < NS - 1)
        def _():
            pltpu.sync_copy(acc.at[pl.ds(s * RPT, RPT)],
                            out.at[pl.ds(s * RPT, RPT)])

        @pl.when(s == NS - 1)
        def _():
            pltpu.sync_copy(acc.at[pl.ds((NS - 1) * RPT, N - (NS - 1) * RPT)],
                            out.at[pl.ds((NS - 1) * RPT, N - (NS - 1) * RPT)])

    @pl.when(c == 0)
    def _():
        run(hs_l, out_l)

    @pl.when(c == 1)
    def _():
        run(hs_r, out_r)


def _mm1_body(x_ref, w_ref, dga_ref, dgb_ref, hsl_ref, hsr_ref, dinv_ref):
    dv = lax.rsqrt(dga_ref[...] + dgb_ref[...] + 1.0)
    t = jnp.dot(x_ref[...], w_ref[...],
                preferred_element_type=jnp.float32)
    hs = t * dv
    hsl_ref[...] = hs[:, :HH]
    hsr_ref[...] = hs[:, HH:]
    dinv_ref[...] = dv


def _mid_body(sl_ref, sr_ref, dinv_ref, b_ref, w_ref, hsl_ref, hsr_ref):
    dv = dinv_ref[...]
    sfull = jnp.concatenate([sl_ref[...], sr_ref[...]], axis=1)
    h = jnp.maximum(sfull * dv + b_ref[...], 0.0)
    hs = jnp.dot(h, w_ref[...], preferred_element_type=jnp.float32) * dv
    hsl_ref[...] = hs[:, :HH]
    hsr_ref[...] = hs[:, HH:]


def _pool_body(sl_ref, sr_ref, dinv_ref, b_ref, batch_ref, psum_ref, cnt_ref):
    i = pl.program_id(0)
    dv = dinv_ref[...]
    sfull = jnp.concatenate([sl_ref[...], sr_ref[...]], axis=1)
    h = jnp.maximum(sfull * dv + b_ref[...], 0.0)
    gid = lax.broadcasted_iota(jnp.int32, (BM, G), 1)
    oh = (batch_ref[...] == gid).astype(jnp.float32)
    ps = lax.dot_general(oh, h, (((0,), (0,)), ((), ())),
                         preferred_element_type=jnp.float32)
    pc = lax.dot_general(oh, jnp.ones((BM, 1), jnp.float32),
                         (((0,), (0,)), ((), ())),
                         preferred_element_type=jnp.float32)

    @pl.when(i == 0)
    def _():
        psum_ref[...] = jnp.zeros_like(psum_ref)
        cnt_ref[...] = jnp.zeros_like(cnt_ref)

    psum_ref[...] += ps
    cnt_ref[...] += pc


def _head_body(psum_ref, cnt_ref, wf1_ref, bf1_ref, wf2_ref, bf2_ref, out_ref):
    mean = psum_ref[...] / jnp.maximum(cnt_ref[...], 1.0)
    g1 = jnp.maximum(
        jnp.dot(mean, wf1_ref[...], preferred_element_type=jnp.float32)
        + bf1_ref[...], 0.0)
    out_ref[...] = (jnp.dot(g1, wf2_ref[...],
                            preferred_element_type=jnp.float32) + bf2_ref[...])


def _row_blocks(*shapes):
    return [pl.BlockSpec(sh, lambda i: (i, 0)) for sh in shapes]


def _const_blocks(*shapes):
    return [pl.BlockSpec(sh, lambda i: (0, 0)) for sh in shapes]


_mm1 = pl.pallas_call(
    _mm1_body,
    grid=(NB,),
    in_specs=_row_blocks((BM, F_IN)) + _const_blocks((F_IN, H))
    + _row_blocks((BM, 1), (BM, 1)),
    out_specs=_row_blocks((BM, HH), (BM, HH), (BM, 1)),
    out_shape=(jax.ShapeDtypeStruct((N, HH), jnp.float32),
               jax.ShapeDtypeStruct((N, HH), jnp.float32),
               jax.ShapeDtypeStruct((N, 1), jnp.float32)),
)

_mid = pl.pallas_call(
    _mid_body,
    grid=(NB,),
    in_specs=_row_blocks((BM, HH), (BM, HH), (BM, 1))
    + _const_blocks((1, H), (H, H)),
    out_specs=_row_blocks((BM, HH), (BM, HH)),
    out_shape=(jax.ShapeDtypeStruct((N, HH), jnp.float32),
               jax.ShapeDtypeStruct((N, HH), jnp.float32)),
)

_pool = pl.pallas_call(
    _pool_body,
    grid=(NB,),
    in_specs=_row_blocks((BM, HH), (BM, HH), (BM, 1))
    + _const_blocks((1, H)) + _row_blocks((BM, 1)),
    out_specs=_const_blocks((G, H), (G, 1)),
    out_shape=(jax.ShapeDtypeStruct((G, H), jnp.float32),
               jax.ShapeDtypeStruct((G, 1), jnp.float32)),
)

_head = pl.pallas_call(
    _head_body,
    grid=(1,),
    in_specs=_const_blocks((G, H), (G, 1), (H, H), (1, H), (H, 1), (1, 1)),
    out_specs=_const_blocks((G, 1))[0],
    out_shape=jax.ShapeDtypeStruct((G, 1), jnp.float32),
)


def kernel(x, edge_index, batch, W1, b1, W2, b2, W3, b3, Wf1, bf1, Wf2, bf2):
    pad = EPAD - E
    src_i = jnp.concatenate(
        [edge_index[0], jnp.zeros((pad,), jnp.int32)]).reshape(NS, K, CH)
    dst_i = jnp.concatenate(
        [edge_index[1], jnp.full((pad,), TRASH, jnp.int32)]).reshape(NS, K, CH)

    deg_a, deg_b = _deg_call(dst_i)
    hs_l, hs_r, dinv = _mm1(x, W1, deg_a.reshape(NPAD, 1)[:N],
                            deg_b.reshape(NPAD, 1)[:N])
    s_l, s_r = _agg_call(hs_l, hs_r, src_i, dst_i)
    hs_l, hs_r = _mid(s_l, s_r, dinv, b1.reshape(1, H), W2)
    s_l, s_r = _agg_call(hs_l, hs_r, src_i, dst_i)
    hs_l, hs_r = _mid(s_l, s_r, dinv, b2.reshape(1, H), W3)
    s_l, s_r = _agg_call(hs_l, hs_r, src_i, dst_i)
    psum, cnt = _pool(s_l, s_r, dinv, b3.reshape(1, H), batch.reshape(N, 1))
    return _head(psum, cnt, Wf1, bf1.reshape(1, H), Wf2, bf2.reshape(1, 1))


# speedup vs baseline: 7.4384x; 1.0832x over previous
import functools

import jax
import jax.numpy as jnp
from jax import lax
from jax.experimental import pallas as pl
from jax.experimental.pallas import tpu as pltpu
from jax.experimental.pallas import tpu_sc as plsc

N = 10000
E = 320000
F_IN = 128
H = 256
HH = 128
G = 64
NC = 2
NS = 16
CH = 128
GC = 8
NG = 20
K = NG * GC
EPAD = NS * K * CH
NPAD = 10240
RPT = NPAD // NS
BM = 400
NB = N // BM

_MESH = plsc.VectorSubcoreMesh(core_axis_name="c", subcore_axis_name="s",
                               num_cores=NC, num_subcores=NS)


@functools.partial(
    pl.kernel,
    out_type=(jax.ShapeDtypeStruct((NPAD,), jnp.float32),
              jax.ShapeDtypeStruct((NPAD,), jnp.float32)),
    mesh=_MESH,
    scratch_types=[
        pltpu.VMEM((NG // NC, GC, CH), jnp.int32),
        pltpu.VMEM((RPT,), jnp.float32),
        pltpu.VMEM((CH,), jnp.float32),
        pltpu.VMEM_SHARED((NPAD,), jnp.float32),
        pltpu.SemaphoreType.DMA,
    ],
)
def _deg_call(dst_hbm, deg_a, deg_b, idxv, zbuf, onev, dacc, sem):
    c = lax.axis_index("c")
    s = lax.axis_index("s")
    for i in range(RPT // 16):
        zbuf[pl.ds(i * 16, 16)] = jnp.zeros((16,), jnp.float32)
    for i in range(CH // 16):
        onev[pl.ds(i * 16, 16)] = jnp.ones((16,), jnp.float32)
    pltpu.sync_copy(zbuf, dacc.at[pl.ds(s * RPT, RPT)])
    pltpu.sync_copy(dst_hbm.at[s, pl.ds(c * (NG // NC), NG // NC)], idxv)
    plsc.subcore_barrier()

    def body(jj, carry):
        descs = []
        for b in range(GC):
            descs.append(pltpu.async_copy(onev, dacc.at[idxv.at[jj, b]], sem,
                                          add=True))
        for d in descs:
            d.wait()
        return carry

    lax.fori_loop(0, NG // NC, body, 0)
    plsc.subcore_barrier()

    @pl.when(c == 0)
    def _():
        pltpu.sync_copy(dacc.at[pl.ds(s * RPT, RPT)],
                        deg_a.at[pl.ds(s * RPT, RPT)])

    @pl.when(c == 1)
    def _():
        pltpu.sync_copy(dacc.at[pl.ds(s * RPT, RPT)],
                        deg_b.at[pl.ds(s * RPT, RPT)])


@functools.partial(
    pl.kernel,
    out_type=(jax.ShapeDtypeStruct((N, HH), jnp.float32),
              jax.ShapeDtypeStruct((N, HH), jnp.float32)),
    mesh=_MESH,
    scratch_types=[
        pltpu.VMEM((2, GC, CH), jnp.int32),
        pltpu.VMEM((2, GC, CH), jnp.int32),
        pltpu.VMEM((2, CH, HH), jnp.float32),
        pltpu.VMEM_SHARED((NPAD, HH), jnp.float32),
        pltpu.SemaphoreType.DMA,
        pltpu.SemaphoreType.DMA,
        pltpu.SemaphoreType.DMA,
    ],
)
def _agg_call(hs_l, hs_r, src_hbm, dst_hbm, out_l, out_r,
              isrc, idst, rows, acc, sem_g, sem_s, sem_i):
    c = lax.axis_index("c")
    s = lax.axis_index("s")

    def run(hs, out):
        @pl.when(s < NS - 1)
        def _():
            pltpu.sync_copy(hs.at[pl.ds(s * RPT, RPT)],
                            acc.at[pl.ds(s * RPT, RPT)])

        @pl.when(s == NS - 1)
        def _():
            pltpu.sync_copy(hs.at[pl.ds((NS - 1) * RPT, N - (NS - 1) * RPT)],
                            acc.at[pl.ds((NS - 1) * RPT, N - (NS - 1) * RPT)])

        plsc.subcore_barrier()

        def idx_start(g, p):
            pltpu.async_copy(src_hbm.at[s, g], isrc.at[p], sem_i)
            pltpu.async_copy(dst_hbm.at[s, g], idst.at[p], sem_i)

        def idx_wait(g, p):
            pltpu.make_async_copy(src_hbm.at[s, g], isrc.at[p], sem_i).wait()
            pltpu.make_async_copy(dst_hbm.at[s, g], idst.at[p], sem_i).wait()

        def gat_start(p, b, rb):
            pltpu.async_copy(hs.at[isrc.at[p, b]], rows.at[rb], sem_g)

        def gat_wait(p, b, rb):
            pltpu.make_async_copy(hs.at[isrc.at[p, b]], rows.at[rb],
                                  sem_g).wait()

        def scat_start(p, b, rb):
            pltpu.async_copy(rows.at[rb], acc.at[idst.at[p, b]], sem_s,
                             add=True)

        def scat_wait(p, b, rb):
            pltpu.make_async_copy(rows.at[rb], acc.at[idst.at[p, b]],
                                  sem_s).wait()

        idx_start(0, 0)

        def group(g, p):
            idx_wait(g, p)
            for b in range(GC):
                k = g * GC + b
                rb = b % 2
                if b == 2:
                    @pl.when(g + 1 < NG)
                    def _():
                        idx_start(g + 1, 1 - p)
                if b >= 2:
                    scat_wait(p, b - 2, rb)
                else:
                    @pl.when(k >= 2)
                    def _():
                        scat_wait(1 - p, b + GC - 2, rb)
                gat_start(p, b, rb)
                if b >= 1:
                    gat_wait(p, b - 1, 1 - rb)
                    scat_start(p, b - 1, 1 - rb)
                else:
                    @pl.when(k >= 1)
                    def _():
                        gat_wait(1 - p, GC - 1, 1 - rb)
                        scat_start(1 - p, GC - 1, 1 - rb)

        def body(gg, carry):
            group(2 * gg, 0)
            group(2 * gg + 1, 1)
            return carry

        lax.fori_loop(0, NG // 2, body, 0)
        gat_wait(1, GC - 1, 1)
        scat_start(1, GC - 1, 1)
        scat_wait(1, GC - 2, 0)
        scat_wait(1, GC - 1, 1)
        plsc.subcore_barrier()

        @pl.when(s < NS - 1)
        def _():
            pltpu.sync_copy(acc.at[pl.ds(s * RPT, RPT)],
                            out.at[pl.ds(s * RPT, RPT)])

        @pl.when(s == NS - 1)
        def _():
            pltpu.sync_copy(acc.at[pl.ds((NS - 1) * RPT, N - (NS - 1) * RPT)],
                            out.at[pl.ds((NS - 1) * RPT, N - (NS - 1) * RPT)])

    @pl.when(c == 0)
    def _():
        run(hs_l, out_l)

    @pl.when(c == 1)
    def _():
        run(hs_r, out_r)


def _mm1_body(x_ref, w_ref, dga_ref, dgb_ref, hsl_ref, hsr_ref, dinv_ref):
    dv = lax.rsqrt(dga_ref[...] + dgb_ref[...] + 1.0)
    t = jnp.dot(x_ref[...], w_ref[...],
                preferred_element_type=jnp.float32)
    hs = t * dv
    hsl_ref[...] = hs[:, :HH]
    hsr_ref[...] = hs[:, HH:]
    dinv_ref[...] = dv


def _mid_body(sl_ref, sr_ref, dinv_ref, b_ref, w_ref, hsl_ref, hsr_ref):
    dv = dinv_ref[...]
    sfull = jnp.concatenate([sl_ref[...], sr_ref[...]], axis=1)
    h = jnp.maximum(sfull * dv + b_ref[...], 0.0)
    hs = jnp.dot(h, w_ref[...], preferred_element_type=jnp.float32) * dv
    hsl_ref[...] = hs[:, :HH]
    hsr_ref[...] = hs[:, HH:]


def _pool_body(sl_ref, sr_ref, dinv_ref, b_ref, batch_ref, psum_ref, cnt_ref):
    i = pl.program_id(0)
    dv = dinv_ref[...]
    sfull = jnp.concatenate([sl_ref[...], sr_ref[...]], axis=1)
    h = jnp.maximum(sfull * dv + b_ref[...], 0.0)
    gid = lax.broadcasted_iota(jnp.int32, (BM, G), 1)
    oh = (batch_ref[...] == gid).astype(jnp.float32)
    ps = lax.dot_general(oh, h, (((0,), (0,)), ((), ())),
                         preferred_element_type=jnp.float32)
    pc = lax.dot_general(oh, jnp.ones((BM, 1), jnp.float32),
                         (((0,), (0,)), ((), ())),
                         preferred_element_type=jnp.float32)

    @pl.when(i == 0)
    def _():
        psum_ref[...] = jnp.zeros_like(psum_ref)
        cnt_ref[...] = jnp.zeros_like(cnt_ref)

    psum_ref[...] += ps
    cnt_ref[...] += pc


def _head_body(psum_ref, cnt_ref, wf1_ref, bf1_ref, wf2_ref, bf2_ref, out_ref):
    mean = psum_ref[...] / jnp.maximum(cnt_ref[...], 1.0)
    g1 = jnp.maximum(
        jnp.dot(mean, wf1_ref[...], preferred_element_type=jnp.float32)
        + bf1_ref[...], 0.0)
    out_ref[...] = (jnp.dot(g1, wf2_ref[...],
                            preferred_element_type=jnp.float32) + bf2_ref[...])


def _row_blocks(*shapes):
    return [pl.BlockSpec(sh, lambda i: (i, 0)) for sh in shapes]


def _const_blocks(*shapes):
    return [pl.BlockSpec(sh, lambda i: (0, 0)) for sh in shapes]


_mm1 = pl.pallas_call(
    _mm1_body,
    grid=(NB,),
    in_specs=_row_blocks((BM, F_IN)) + _const_blocks((F_IN, H))
    + _row_blocks((BM, 1), (BM, 1)),
    out_specs=_row_blocks((BM, HH), (BM, HH), (BM, 1)),
    out_shape=(jax.ShapeDtypeStruct((N, HH), jnp.float32),
               jax.ShapeDtypeStruct((N, HH), jnp.float32),
               jax.ShapeDtypeStruct((N, 1), jnp.float32)),
)

_mid = pl.pallas_call(
    _mid_body,
    grid=(NB,),
    in_specs=_row_blocks((BM, HH), (BM, HH), (BM, 1))
    + _const_blocks((1, H), (H, H)),
    out_specs=_row_blocks((BM, HH), (BM, HH)),
    out_shape=(jax.ShapeDtypeStruct((N, HH), jnp.float32),
               jax.ShapeDtypeStruct((N, HH), jnp.float32)),
)

_pool = pl.pallas_call(
    _pool_body,
    grid=(NB,),
    in_specs=_row_blocks((BM, HH), (BM, HH), (BM, 1))
    + _const_blocks((1, H)) + _row_blocks((BM, 1)),
    out_specs=_const_blocks((G, H), (G, 1)),
    out_shape=(jax.ShapeDtypeStruct((G, H), jnp.float32),
               jax.ShapeDtypeStruct((G, 1), jnp.float32)),
)

_head = pl.pallas_call(
    _head_body,
    grid=(1,),
    in_specs=_const_blocks((G, H), (G, 1), (H, H), (1, H), (H, 1), (1, 1)),
    out_specs=_const_blocks((G, 1))[0],
    out_shape=jax.ShapeDtypeStruct((G, 1), jnp.float32),
)


def kernel(x, edge_index, batch, W1, b1, W2, b2, W3, b3, Wf1, bf1, Wf2, bf2):
    pad = EPAD - E
    src_i = jnp.concatenate(
        [edge_index[0], jnp.zeros((pad,), jnp.int32)]).reshape(NS, NG, GC, CH)
    pad_dst = N + (jnp.arange(pad, dtype=jnp.int32) % 16)
    dst_i = jnp.concatenate([edge_index[1], pad_dst]).reshape(NS, NG, GC, CH)

    deg_a, deg_b = _deg_call(dst_i)
    hs_l, hs_r, dinv = _mm1(x, W1, deg_a.reshape(NPAD, 1)[:N],
                            deg_b.reshape(NPAD, 1)[:N])
    s_l, s_r = _agg_call(hs_l, hs_r, src_i, dst_i)
    hs_l, hs_r = _mid(s_l, s_r, dinv, b1.reshape(1, H), W2)
    s_l, s_r = _agg_call(hs_l, hs_r, src_i, dst_i)
    hs_l, hs_r = _mid(s_l, s_r, dinv, b2.reshape(1, H), W3)
    s_l, s_r = _agg_call(hs_l, hs_r, src_i, dst_i)
    psum, cnt = _pool(s_l, s_r, dinv, b3.reshape(1, H), batch.reshape(N, 1))
    return _head(psum, cnt, Wf1, bf1.reshape(1, H), Wf2, bf2.reshape(1, 1))


# speedup vs baseline: 7.7893x; 1.0472x over previous
import functools

import jax
import jax.numpy as jnp
from jax import lax
from jax.experimental import pallas as pl
from jax.experimental.pallas import tpu as pltpu
from jax.experimental.pallas import tpu_sc as plsc

N = 10000
E = 320000
F_IN = 128
H = 256
HH = 128
G = 64
NC = 2
NS = 16
CH = 128
GC = 8
NG = 20
K = NG * GC
EPAD = NS * K * CH
NPAD = 10240
RPT = NPAD // NS
BM = 400
NB = N // BM

_MESH = plsc.VectorSubcoreMesh(core_axis_name="c", subcore_axis_name="s",
                               num_cores=NC, num_subcores=NS)


@functools.partial(
    pl.kernel,
    out_type=(jax.ShapeDtypeStruct((NPAD,), jnp.float32),
              jax.ShapeDtypeStruct((NPAD,), jnp.float32)),
    mesh=_MESH,
    scratch_types=[
        pltpu.VMEM((NG // NC, GC, CH), jnp.int32),
        pltpu.VMEM((RPT,), jnp.float32),
        pltpu.VMEM((CH,), jnp.float32),
        pltpu.VMEM_SHARED((NPAD,), jnp.float32),
        pltpu.SemaphoreType.DMA,
    ],
)
def _deg_call(dst_hbm, deg_a, deg_b, idxv, zbuf, onev, dacc, sem):
    c = lax.axis_index("c")
    s = lax.axis_index("s")
    for i in range(RPT // 16):
        zbuf[pl.ds(i * 16, 16)] = jnp.zeros((16,), jnp.float32)
    for i in range(CH // 16):
        onev[pl.ds(i * 16, 16)] = jnp.ones((16,), jnp.float32)
    pltpu.sync_copy(zbuf, dacc.at[pl.ds(s * RPT, RPT)])
    pltpu.sync_copy(dst_hbm.at[s, pl.ds(c * (NG // NC), NG // NC)], idxv)
    plsc.subcore_barrier()

    def body(jj, carry):
        descs = []
        for b in range(GC):
            descs.append(pltpu.async_copy(onev, dacc.at[idxv.at[jj, b]], sem,
                                          add=True))
        for d in descs:
            d.wait()
        return carry

    lax.fori_loop(0, NG // NC, body, 0)
    plsc.subcore_barrier()

    @pl.when(c == 0)
    def _():
        pltpu.sync_copy(dacc.at[pl.ds(s * RPT, RPT)],
                        deg_a.at[pl.ds(s * RPT, RPT)])

    @pl.when(c == 1)
    def _():
        pltpu.sync_copy(dacc.at[pl.ds(s * RPT, RPT)],
                        deg_b.at[pl.ds(s * RPT, RPT)])


@functools.partial(
    pl.kernel,
    out_type=(jax.ShapeDtypeStruct((N, HH), jnp.float32),
              jax.ShapeDtypeStruct((N, HH), jnp.float32)),
    mesh=_MESH,
    scratch_types=[
        pltpu.VMEM((2, GC, CH), jnp.int32),
        pltpu.VMEM((2, GC, CH), jnp.int32),
        pltpu.VMEM((2, CH, HH), jnp.float32),
        pltpu.VMEM_SHARED((NPAD, HH), jnp.float32),
        pltpu.SemaphoreType.DMA,
        pltpu.SemaphoreType.DMA,
        pltpu.SemaphoreType.DMA,
    ],
)
def _agg_call(hs_l, hs_r, src_hbm, dst_hbm, out_l, out_r,
              isrc, idst, rows, acc, sem_g, sem_s, sem_i):
    c = lax.axis_index("c")
    s = lax.axis_index("s")

    def run(hs, out):
        @pl.when(s < NS - 1)
        def _():
            pltpu.sync_copy(hs.at[pl.ds(s * RPT, RPT)],
                            acc.at[pl.ds(s * RPT, RPT)])

        @pl.when(s == NS - 1)
        def _():
            pltpu.sync_copy(hs.at[pl.ds((NS - 1) * RPT, N - (NS - 1) * RPT)],
                            acc.at[pl.ds((NS - 1) * RPT, N - (NS - 1) * RPT)])

        plsc.subcore_barrier()

        def idx_start(g, p):
            pltpu.async_copy(src_hbm.at[s, g], isrc.at[p], sem_i)
            pltpu.async_copy(dst_hbm.at[s, g], idst.at[p], sem_i)

        def idx_wait(g, p):
            pltpu.make_async_copy(src_hbm.at[s, g], isrc.at[p], sem_i).wait()
            pltpu.make_async_copy(dst_hbm.at[s, g], idst.at[p], sem_i).wait()

        def gat_start(p, b, rb):
            pltpu.async_copy(hs.at[isrc.at[p, b]], rows.at[rb], sem_g)

        def gat_wait(p, b, rb):
            pltpu.make_async_copy(hs.at[isrc.at[p, b]], rows.at[rb],
                                  sem_g).wait()

        def scat_start(p, b, rb):
            pltpu.async_copy(rows.at[rb], acc.at[idst.at[p, b]], sem_s,
                             add=True)

        def scat_wait(p, b, rb):
            pltpu.make_async_copy(rows.at[rb], acc.at[idst.at[p, b]],
                                  sem_s).wait()

        idx_start(0, 0)

        def group(g, p):
            idx_wait(g, p)
            for b in range(GC):
                k = g * GC + b
                rb = b % 2
                if b == 2:
                    @pl.when(g + 1 < NG)
                    def _():
                        idx_start(g + 1, 1 - p)
                if b >= 2:
                    scat_wait(p, b - 2, rb)
                else:
                    @pl.when(k >= 2)
                    def _():
                        scat_wait(1 - p, b + GC - 2, rb)
                gat_start(p, b, rb)
                if b >= 1:
                    gat_wait(p, b - 1, 1 - rb)
                    scat_start(p, b - 1, 1 - rb)
                else:
                    @pl.when(k >= 1)
                    def _():
                        gat_wait(1 - p, GC - 1, 1 - rb)
                        scat_start(1 - p, GC - 1, 1 - rb)

        def body(gg, carry):
            group(2 * gg, 0)
            group(2 * gg + 1, 1)
            return carry

        lax.fori_loop(0, NG // 2, body, 0)
        gat_wait(1, GC - 1, 1)
        scat_start(1, GC - 1, 1)
        scat_wait(1, GC - 2, 0)
        scat_wait(1, GC - 1, 1)
        plsc.subcore_barrier()

        @pl.when(s < NS - 1)
        def _():
            pltpu.sync_copy(acc.at[pl.ds(s * RPT, RPT)],
                            out.at[pl.ds(s * RPT, RPT)])

        @pl.when(s == NS - 1)
        def _():
            pltpu.sync_copy(acc.at[pl.ds((NS - 1) * RPT, N - (NS - 1) * RPT)],
                            out.at[pl.ds((NS - 1) * RPT, N - (NS - 1) * RPT)])

    @pl.when(c == 0)
    def _():
        run(hs_r, out_r)

    @pl.when(c == 1)
    def _():
        run(hs_l, out_l)


def _mm1_body(x_ref, w_ref, dga_ref, dgb_ref, hsl_ref, hsr_ref, dinv_ref):
    dv = lax.rsqrt(dga_ref[...] + dgb_ref[...] + 1.0)
    t = jnp.dot(x_ref[...], w_ref[...],
                preferred_element_type=jnp.float32)
    hs = t * dv
    hsl_ref[...] = hs[:, :HH]
    hsr_ref[...] = hs[:, HH:]
    dinv_ref[...] = dv


def _mid_body(sl_ref, sr_ref, dinv_ref, b_ref, w_ref, hsl_ref, hsr_ref):
    dv = dinv_ref[...]
    sfull = jnp.concatenate([sl_ref[...], sr_ref[...]], axis=1)
    h = jnp.maximum(sfull * dv + b_ref[...], 0.0)
    hs = jnp.dot(h, w_ref[...], preferred_element_type=jnp.float32) * dv
    hsl_ref[...] = hs[:, :HH]
    hsr_ref[...] = hs[:, HH:]


def _pool_body(sl_ref, sr_ref, dinv_ref, b_ref, batch_ref, psum_ref, cnt_ref):
    i = pl.program_id(0)
    dv = dinv_ref[...]
    sfull = jnp.concatenate([sl_ref[...], sr_ref[...]], axis=1)
    h = jnp.maximum(sfull * dv + b_ref[...], 0.0)
    gid = lax.broadcasted_iota(jnp.int32, (BM, G), 1)
    oh = (batch_ref[...] == gid).astype(jnp.float32)
    ps = lax.dot_general(oh, h, (((0,), (0,)), ((), ())),
                         preferred_element_type=jnp.float32)
    pc = lax.dot_general(oh, jnp.ones((BM, 1), jnp.float32),
                         (((0,), (0,)), ((), ())),
                         preferred_element_type=jnp.float32)

    @pl.when(i == 0)
    def _():
        psum_ref[...] = jnp.zeros_like(psum_ref)
        cnt_ref[...] = jnp.zeros_like(cnt_ref)

    psum_ref[...] += ps
    cnt_ref[...] += pc


def _head_body(psum_ref, cnt_ref, wf1_ref, bf1_ref, wf2_ref, bf2_ref, out_ref):
    mean = psum_ref[...] / jnp.maximum(cnt_ref[...], 1.0)
    g1 = jnp.maximum(
        jnp.dot(mean, wf1_ref[...], preferred_element_type=jnp.float32)
        + bf1_ref[...], 0.0)
    out_ref[...] = (jnp.dot(g1, wf2_ref[...],
                            preferred_element_type=jnp.float32) + bf2_ref[...])


def _row_blocks(*shapes):
    return [pl.BlockSpec(sh, lambda i: (i, 0)) for sh in shapes]


def _const_blocks(*shapes):
    return [pl.BlockSpec(sh, lambda i: (0, 0)) for sh in shapes]


_mm1 = pl.pallas_call(
    _mm1_body,
    grid=(NB,),
    in_specs=_row_blocks((BM, F_IN)) + _const_blocks((F_IN, H))
    + _row_blocks((BM, 1), (BM, 1)),
    out_specs=_row_blocks((BM, HH), (BM, HH), (BM, 1)),
    out_shape=(jax.ShapeDtypeStruct((N, HH), jnp.float32),
               jax.ShapeDtypeStruct((N, HH), jnp.float32),
               jax.ShapeDtypeStruct((N, 1), jnp.float32)),
)

_mid = pl.pallas_call(
    _mid_body,
    grid=(NB,),
    in_specs=_row_blocks((BM, HH), (BM, HH), (BM, 1))
    + _const_blocks((1, H), (H, H)),
    out_specs=_row_blocks((BM, HH), (BM, HH)),
    out_shape=(jax.ShapeDtypeStruct((N, HH), jnp.float32),
               jax.ShapeDtypeStruct((N, HH), jnp.float32)),
)

_pool = pl.pallas_call(
    _pool_body,
    grid=(NB,),
    in_specs=_row_blocks((BM, HH), (BM, HH), (BM, 1))
    + _const_blocks((1, H)) + _row_blocks((BM, 1)),
    out_specs=_const_blocks((G, H), (G, 1)),
    out_shape=(jax.ShapeDtypeStruct((G, H), jnp.float32),
               jax.ShapeDtypeStruct((G, 1), jnp.float32)),
)

_head = pl.pallas_call(
    _head_body,
    grid=(1,),
    in_specs=_const_blocks((G, H), (G, 1), (H, H), (1, H), (H, 1), (1, 1)),
    out_specs=_const_blocks((G, 1))[0],
    out_shape=jax.ShapeDtypeStruct((G, 1), jnp.float32),
)


def kernel(x, edge_index, batch, W1, b1, W2, b2, W3, b3, Wf1, bf1, Wf2, bf2):
    pad = EPAD - E
    src_i = jnp.concatenate(
        [edge_index[0], jnp.zeros((pad,), jnp.int32)]).reshape(NS, NG, GC, CH)
    pad_dst = N + (jnp.arange(pad, dtype=jnp.int32) % 16)
    dst_i = jnp.concatenate([edge_index[1], pad_dst]).reshape(NS, NG, GC, CH)

    deg_a, deg_b = _deg_call(dst_i)
    hs_l, hs_r, dinv = _mm1(x, W1, deg_a.reshape(NPAD, 1)[:N],
                            deg_b.reshape(NPAD, 1)[:N])
    s_l, s_r = _agg_call(hs_l, hs_r, src_i, dst_i)
    hs_l, hs_r = _mid(s_l, s_r, dinv, b1.reshape(1, H), W2)
    s_l, s_r = _agg_call(hs_l, hs_r, src_i, dst_i)
    hs_l, hs_r = _mid(s_l, s_r, dinv, b2.reshape(1, H), W3)
    s_l, s_r = _agg_call(hs_l, hs_r, src_i, dst_i)
    psum, cnt = _pool(s_l, s_r, dinv, b3.reshape(1, H), batch.reshape(N, 1))
    return _head(psum, cnt, Wf1, bf1.reshape(1, H), Wf2, bf2.reshape(1, 1))
